# baseline (speedup 1.0000x reference)
import functools
import jax
import jax.numpy as jnp
from jax import lax
from jax.experimental import pallas as pl
from jax.experimental.pallas import tpu as pltpu
from jax.experimental.pallas import tpu_sc as plsc

D_MODEL = 2048
NUM_ROWS = 32

_info = plsc.get_sparse_core_info()
_NC, _NS = _info.num_cores, _info.num_subcores
_NW = _NC * _NS


@functools.partial(jax.jit, static_argnames=("n", "chunk"))
def _emb_lookup(ids_flat, emb_weight, n, chunk):
    b_per_w = n // _NW
    n_chunks = b_per_w // chunk
    mesh = plsc.VectorSubcoreMesh(core_axis_name="c", subcore_axis_name="s")

    @functools.partial(
        pl.kernel,
        mesh=mesh,
        out_type=jax.ShapeDtypeStruct((n, D_MODEL), jnp.float32),
        scratch_types=[
            pltpu.VMEM((b_per_w,), jnp.int32),
            pltpu.VMEM((chunk, D_MODEL), jnp.float32),
            pltpu.VMEM((chunk, D_MODEL), jnp.float32),
            pltpu.SemaphoreType.DMA,
            pltpu.SemaphoreType.DMA,
        ],
    )
    def k(table_hbm, ids_hbm, out_hbm, idx_v, buf0, buf1, sem0, sem1):
        wid = lax.axis_index("s") * _NC + lax.axis_index("c")
        base = wid * b_per_w
        pltpu.sync_copy(ids_hbm.at[pl.ds(base, b_per_w)], idx_v)

        bufs = (buf0, buf1)
        sems = (sem0, sem1)

        pltpu.async_copy(table_hbm.at[idx_v.at[pl.ds(0, chunk)]], buf0, sem0)

        def body(c, _):
            slot = lax.rem(c, 2)
            nxt = c + 1

            @pl.when(nxt < n_chunks)
            def _():
                nslot = lax.rem(nxt, 2)
                for s in range(2):

                    @pl.when(nslot == s)
                    def _():
                        pltpu.async_copy(
                            table_hbm.at[idx_v.at[pl.ds(nxt * chunk, chunk)]],
                            bufs[s],
                            sems[s],
                        )

            for s in range(2):

                @pl.when(slot == s)
                def _():
                    pltpu.make_async_copy(
                        table_hbm.at[idx_v.at[pl.ds(c * chunk, chunk)]],
                        bufs[s],
                        sems[s],
                    ).wait()
                    pltpu.sync_copy(
                        bufs[s], out_hbm.at[pl.ds(base + c * chunk, chunk)]
                    )

            return ()

        lax.fori_loop(0, n_chunks, body, (), unroll=False)

    return k(emb_weight, ids_flat)


def kernel(cont_ids, emb_weight):
    b, t = cont_ids.shape
    n = b * t
    ids_flat = cont_ids.reshape(n).astype(jnp.int32)
    out = _emb_lookup(ids_flat, emb_weight, n, 16)
    return out.reshape(b, t, D_MODEL)


# speedup vs baseline: 3.7302x; 3.7302x over previous
import functools
import jax
import jax.numpy as jnp
from jax import lax
from jax.experimental import pallas as pl
from jax.experimental.pallas import tpu as pltpu
from jax.experimental.pallas import tpu_sc as plsc

D_MODEL = 2048
NUM_ROWS = 32

_info = plsc.get_sparse_core_info()
_NC, _NS = _info.num_cores, _info.num_subcores
_NW = _NC * _NS


@functools.partial(jax.jit, static_argnames=("n",))
def _emb_lookup(ids_flat, emb_weight, n):
    b_per_w = n // _NW
    U = 16
    W = 4
    n_ch = b_per_w // U
    mesh = plsc.VectorSubcoreMesh(core_axis_name="c", subcore_axis_name="s")

    @functools.partial(
        pl.kernel,
        mesh=mesh,
        out_type=jax.ShapeDtypeStruct((n, D_MODEL), jnp.float32),
        scratch_types=[
            pltpu.VMEM((NUM_ROWS, D_MODEL), jnp.float32),
            pltpu.VMEM((b_per_w,), jnp.int32),
            pltpu.SemaphoreType.DMA,
        ],
    )
    def k(table_hbm, ids_hbm, out_hbm, table_v, idx_v, sem):
        wid = lax.axis_index("s") * _NC + lax.axis_index("c")
        base = wid * b_per_w
        pltpu.sync_copy(table_hbm, table_v)
        pltpu.sync_copy(ids_hbm.at[pl.ds(base, b_per_w)], idx_v)

        def wait_chunk():
            pltpu.make_async_copy(
                table_v.at[pl.ds(0, U)], out_hbm.at[pl.ds(base, U)], sem
            ).wait()

        def body(c, _):
            i0 = c * U
            ids_vec = idx_v[pl.ds(i0, U)]
            for j in range(U):
                row = ids_vec[j]
                pltpu.async_copy(
                    table_v.at[pl.ds(row, 1)],
                    out_hbm.at[pl.ds(base + i0 + j, 1)],
                    sem,
                )

            @pl.when(c >= W)
            def _():
                wait_chunk()

            return ()

        lax.fori_loop(0, n_ch, body, (), unroll=False)

        def dbody(c, _):
            wait_chunk()
            return ()

        lax.fori_loop(0, W, dbody, (), unroll=False)

    return k(emb_weight, ids_flat)


def kernel(cont_ids, emb_weight):
    b, t = cont_ids.shape
    n = b * t
    ids_flat = cont_ids.reshape(n).astype(jnp.int32)
    out = _emb_lookup(ids_flat, emb_weight, n)
    return out.reshape(b, t, D_MODEL)


# speedup vs baseline: 3.7362x; 1.0016x over previous
import functools
import jax
import jax.numpy as jnp
from jax import lax
from jax.experimental import pallas as pl
from jax.experimental.pallas import tpu as pltpu
from jax.experimental.pallas import tpu_sc as plsc

D_MODEL = 2048
NUM_ROWS = 32

_info = plsc.get_sparse_core_info()
_NC, _NS = _info.num_cores, _info.num_subcores
_NW = _NC * _NS


@functools.partial(jax.jit, static_argnames=("n",))
def _emb_lookup(ids_flat, emb_weight, n):
    b_per_w = n // _NW
    U = 16
    W = 8
    n_ch = b_per_w // U
    mesh = plsc.VectorSubcoreMesh(core_axis_name="c", subcore_axis_name="s")

    @functools.partial(
        pl.kernel,
        mesh=mesh,
        out_type=jax.ShapeDtypeStruct((n, D_MODEL), jnp.float32),
        scratch_types=[
            pltpu.VMEM((NUM_ROWS, D_MODEL), jnp.float32),
            pltpu.VMEM((b_per_w,), jnp.int32),
            pltpu.SemaphoreType.DMA,
        ],
    )
    def k(table_hbm, ids_hbm, out_hbm, table_v, idx_v, sem):
        wid = lax.axis_index("s") * _NC + lax.axis_index("c")
        base = wid * b_per_w
        pltpu.sync_copy(table_hbm, table_v)
        pltpu.sync_copy(ids_hbm.at[pl.ds(base, b_per_w)], idx_v)

        def wait_chunk():
            pltpu.make_async_copy(
                table_v.at[pl.ds(0, U)], out_hbm.at[pl.ds(base, U)], sem
            ).wait()

        def body(c, _):
            i0 = c * U
            ids_vec = idx_v[pl.ds(i0, U)]
            for j in range(U):
                row = ids_vec[j]
                pltpu.async_copy(
                    table_v.at[pl.ds(row, 1)],
                    out_hbm.at[pl.ds(base + i0 + j, 1)],
                    sem,
                )

            @pl.when(c >= W)
            def _():
                wait_chunk()

            return ()

        lax.fori_loop(0, n_ch, body, (), unroll=False)

        def dbody(c, _):
            wait_chunk()
            return ()

        lax.fori_loop(0, W, dbody, (), unroll=False)

    return k(emb_weight, ids_flat)


def kernel(cont_ids, emb_weight):
    b, t = cont_ids.shape
    n = b * t
    ids_flat = cont_ids.reshape(n).astype(jnp.int32)
    out = _emb_lookup(ids_flat, emb_weight, n)
    return out.reshape(b, t, D_MODEL)


# speedup vs baseline: 4.5908x; 1.2287x over previous
import functools
import jax
import jax.numpy as jnp
from jax import lax
from jax.experimental import pallas as pl
from jax.experimental.pallas import tpu as pltpu
from jax.experimental.pallas import tpu_sc as plsc

D_MODEL = 2048
NUM_ROWS = 32

_info = plsc.get_sparse_core_info()
_NC, _NS = _info.num_cores, _info.num_subcores
_NW = _NC * _NS


@functools.partial(jax.jit, static_argnames=("n",))
def _emb_lookup(ids_flat, emb_weight, n):
    b_per_w = n // _NW
    U = 16
    W = 8
    n_ch = b_per_w // U
    mesh = plsc.VectorSubcoreMesh(core_axis_name="c", subcore_axis_name="s")

    @functools.partial(
        pl.kernel,
        mesh=mesh,
        out_type=jax.ShapeDtypeStruct((n, D_MODEL), jnp.float32),
        scratch_types=[
            pltpu.VMEM((NUM_ROWS, D_MODEL), jnp.float32),
            pltpu.VMEM((b_per_w,), jnp.int32),
            pltpu.SemaphoreType.DMA,
        ],
    )
    def k(table_hbm, ids_hbm, out_hbm, table_v, idx_v, sem):
        wid = lax.axis_index("s") * _NC + lax.axis_index("c")
        base = wid * b_per_w
        pltpu.sync_copy(table_hbm, table_v)
        pltpu.sync_copy(ids_hbm.at[pl.ds(base, b_per_w)], idx_v)

        def wait_chunk():
            pltpu.make_async_copy(
                table_v.at[pl.ds(0, U)], out_hbm.at[pl.ds(base, U)], sem
            ).wait()

        def body(c, _):
            i0 = c * U
            ids_vec = idx_v[pl.ds(i0, U)]
            for j in range(U):
                row = ids_vec[j]
                pltpu.async_copy(
                    table_v.at[pl.ds(row, 1)],
                    out_hbm.at[pl.ds(base + i0 + j, 1)],
                    sem,
                )

            @pl.when(c >= W)
            def _():
                wait_chunk()

            return ()

        lax.fori_loop(0, n_ch, body, (), unroll=False)

        def dbody(c, _):
            wait_chunk()
            return ()

        lax.fori_loop(0, W, dbody, (), unroll=False)

    return k(emb_weight, ids_flat)


@functools.partial(jax.jit, static_argnames=("n", "blk"))
def _tc_lookup(ids_flat, emb_weight, n, blk):
    nb = n // blk
    ids3 = ids_flat.reshape(nb, 1, blk)

    def body(ids_ref, tab_ref, out_ref):
        ids = ids_ref[0, 0, :]
        oh = (
            ids[:, None]
            == lax.broadcasted_iota(jnp.int32, (blk, NUM_ROWS), 1)
        ).astype(jnp.float32)
        out_ref[...] = jnp.dot(
            oh, tab_ref[...], preferred_element_type=jnp.float32
        )

    return pl.pallas_call(
        body,
        grid=(nb,),
        in_specs=[
            pl.BlockSpec((1, 1, blk), lambda i: (i, 0, 0)),
            pl.BlockSpec((NUM_ROWS, D_MODEL), lambda i: (0, 0)),
        ],
        out_specs=pl.BlockSpec((blk, D_MODEL), lambda i: (i, 0)),
        out_shape=jax.ShapeDtypeStruct((n, D_MODEL), jnp.float32),
    )(ids3, emb_weight)


def kernel(cont_ids, emb_weight):
    b, t = cont_ids.shape
    n = b * t
    ids_flat = cont_ids.reshape(n).astype(jnp.int32)
    out = _tc_lookup(ids_flat, emb_weight, n, 512)
    return out.reshape(b, t, D_MODEL)
